# Initial kernel scaffold; baseline (speedup 1.0000x reference)
import jax
import jax.numpy as jnp
from jax.experimental import pallas as pl


def kernel(synd, converge, llr, H_matrix, e_v):
    raise NotImplementedError("write your pallas kernel here")


# speedup vs baseline: 33.9735x; 33.9735x over previous
import jax
import jax.numpy as jnp
from jax import lax
from jax.experimental import pallas as pl
from jax.experimental.pallas import tpu as pltpu

_B, _M, _N = 16, 256, 512
_W = 8


def _body(synd_t_ref, conv_col_ref, conv_row_ref, llr_ref, llrt_ref, h_ref,
          evt_ref, e_out_ref, mi_ref, a_scr, used_scr, cnt_scr, ispiv_scr):
    B, M, N, W = _B, _M, _N, _W

    q_i = lax.broadcasted_iota(jnp.int32, (4 * W, M), 0)
    r_i = lax.broadcasted_iota(jnp.int32, (4 * W, M), 1)
    g = q_i // W
    w = q_i % W
    k = r_i - (32 * w + 8 * g)
    active = (k >= 0) & (k < 8)
    bmat = jnp.where(active, jnp.int32(1) << jnp.clip(k, 0, 7), 0)
    bmat_f = bmat.astype(jnp.float32)

    h_bytes = jnp.dot(bmat_f, h_ref[...], preferred_element_type=jnp.float32)
    s_bytes = jnp.dot(bmat_f, synd_t_ref[...],
                      preferred_element_type=jnp.float32).astype(jnp.int32)

    iota_lane_nn = lax.broadcasted_iota(jnp.int32, (N, N), 1)
    iota_row_nn = lax.broadcasted_iota(jnp.int32, (N, N), 0)

    ranks = []
    for b in range(B):
        xrow = llr_ref[b:b + 1, :]
        xcol = llrt_ref[:, b:b + 1]
        lt = xrow < xcol
        tie = (xrow == xcol) & (iota_lane_nn < iota_row_nn)
        cmp = (lt | tie).astype(jnp.int32)
        rank2 = jnp.sum(cmp, axis=1, keepdims=True)
        ranks.append(rank2)
        p_f = (rank2 == iota_lane_nn).astype(jnp.float32)
        ab = jnp.dot(h_bytes, p_f,
                     preferred_element_type=jnp.float32).astype(jnp.int32)
        words = (ab[0:W] | (ab[W:2 * W] << 8) | (ab[2 * W:3 * W] << 16)
                 | (ab[3 * W:4 * W] << 24))
        a_scr[b, :, 0:N] = words
        sb = s_bytes[:, b:b + 1]
        a_scr[b, :, N:N + 1] = (sb[0:W] | (sb[W:2 * W] << 8)
                                | (sb[2 * W:3 * W] << 16)
                                | (sb[3 * W:4 * W] << 24))

    iota_col = lax.broadcasted_iota(jnp.int32, (B, W, N + 1), 2)
    iota_w = lax.broadcasted_iota(jnp.int32, (B, W, 1), 1)
    iota_ip = lax.broadcasted_iota(jnp.int32, (B, N + 1), 1)

    used_scr[...] = jnp.zeros((B, W, 1), jnp.int32)
    cnt_scr[...] = jnp.zeros((B, 1), jnp.int32)
    ispiv_scr[...] = jnp.zeros((B, N + 1), jnp.int32)

    def elim(j, carry):
        used = used_scr[...]
        a = a_scr[...]
        col = jnp.sum(jnp.where(iota_col == j, a, 0), axis=2, keepdims=True)
        avail = col & ~used
        nzw = avail != 0
        widx = jnp.min(jnp.where(nzw, iota_w, W), axis=1, keepdims=True)
        has = widx < W
        lsb = avail & (-avail)
        piv = jnp.where(iota_w == widx, lsb, 0)
        mask = jnp.where(has, col ^ piv, 0)
        tst = jnp.any((a & piv) != 0, axis=1, keepdims=True)
        a_scr[...] = a ^ jnp.where(tst, mask, 0)
        used_scr[...] = used | piv
        cnt_scr[...] = cnt_scr[...] + has.astype(jnp.int32).reshape(B, 1)
        ispiv_scr[...] = ispiv_scr[...] | (
            (iota_ip == j) & has.reshape(B, 1)).astype(jnp.int32)
        return carry

    lax.fori_loop(0, N, elim, 0)
    cnt = cnt_scr[...]
    is_piv = ispiv_scr[...] != 0

    a = a_scr[...]
    syndcol = a[:, :, N:N + 1]
    ep = jnp.any((a[:, :, 0:N] & syndcol) != 0, axis=1) & is_piv[:, 0:N]
    epi = ep.astype(jnp.int32)

    ncv_col = conv_col_ref[...] == 0
    mi = jnp.max(jnp.where(ncv_col, cnt, 0))
    mi_ref[...] = jnp.broadcast_to(mi, (1, 1))

    for b in range(B):
        p_onehot = ranks[b] == iota_lane_nn
        eprow = epi[b:b + 1, :]
        e_col = jnp.sum(jnp.where(p_onehot, eprow, 0), axis=1, keepdims=True)
        e_out_ref[:, b:b + 1] = e_col

    eo = e_out_ref[...]
    e_out_ref[...] = jnp.where(conv_row_ref[...] == 0, eo, evt_ref[...])


def kernel(synd, converge, llr, H_matrix, e_v):
    B, N = llr.shape
    synd_t = synd.astype(jnp.float32).T
    conv_col = converge.astype(jnp.int32).reshape(B, 1)
    conv_row = converge.astype(jnp.int32).reshape(1, B)
    h_f = H_matrix.astype(jnp.float32)
    llr_t = llr.T
    evt = e_v.astype(jnp.uint8).astype(jnp.int32).T

    e_t, mi = pl.pallas_call(
        _body,
        out_shape=[
            jax.ShapeDtypeStruct((N, B), jnp.int32),
            jax.ShapeDtypeStruct((1, 1), jnp.int32),
        ],
        scratch_shapes=[
            pltpu.VMEM((B, _W, N + 1), jnp.int32),
            pltpu.VMEM((B, _W, 1), jnp.int32),
            pltpu.VMEM((B, 1), jnp.int32),
            pltpu.VMEM((B, N + 1), jnp.int32),
        ],
    )(synd_t, conv_col, conv_row, llr, llr_t, h_f, evt)

    final = e_t.T.astype(jnp.uint8)
    return final, mi[0, 0], jnp.ones_like(converge)
